# baseline (speedup 1.0000x reference)
import jax
import jax.numpy as jnp
from jax import lax
from jax.experimental import pallas as pl
from jax.experimental.pallas import tpu as pltpu
from jax.experimental.pallas import tpu_sc as plsc

B = 2048
D = 2048
E = 8
NC = 16
FB = 16
FPE = 2
H = 1024
RH = 1024
K = 2

T = 256
MAX_TILES = 23
P = MAX_TILES * T


_SQRT_HALF = 0.7071067811865476


def _gelu(x):
    return 0.5 * x * (1.0 + jax.lax.erf(x * _SQRT_HALF))


def _router_body(hid_ref, sf_ref, g_ref, b_ref, w1h_ref, w1f_ref, b1_ref,
                 w2_ref, b2_ref, h_out_ref, r4_ref):
    x = hid_ref[...]
    mu = jnp.mean(x, axis=1, keepdims=True)
    xc = x - mu
    var = jnp.mean(xc * xc, axis=1, keepdims=True)
    h = xc * jax.lax.rsqrt(var + 1e-5) * g_ref[...] + b_ref[...]
    h_out_ref[...] = h
    t1 = jnp.dot(h, w1h_ref[...], preferred_element_type=jnp.float32)
    t1 = t1 + jnp.dot(sf_ref[...], w1f_ref[...],
                      preferred_element_type=jnp.float32)
    t1 = _gelu(t1 + b1_ref[...])
    logits = jnp.dot(t1, w2_ref[...],
                     preferred_element_type=jnp.float32) + b2_ref[...]
    i1 = jnp.argmax(logits, axis=1)
    v1 = jnp.max(logits, axis=1, keepdims=True)
    masked = jnp.where(jnp.arange(E)[None, :] == i1[:, None],
                       -jnp.inf, logits)
    i2 = jnp.argmax(masked, axis=1)
    v2 = jnp.max(masked, axis=1, keepdims=True)
    e2 = jnp.exp(v2 - v1)
    w1 = 1.0 / (1.0 + e2)
    w2 = e2 * w1
    r4_ref[...] = jnp.concatenate(
        [i1[:, None].astype(jnp.float32), i2[:, None].astype(jnp.float32),
         w1, w2], axis=1)


def _run_router(hidden, stage_feats, ln_gamma, ln_beta, rW1, rb1, rW2, rb2):
    TB = 256
    grid = (B // TB,)
    w1h = rW1[:D]
    w1f = rW1[D:]
    h_ln, r4 = pl.pallas_call(
        _router_body,
        grid=grid,
        in_specs=[
            pl.BlockSpec((TB, D), lambda i: (i, 0)),
            pl.BlockSpec((TB, NC * FB), lambda i: (i, 0)),
            pl.BlockSpec((1, D), lambda i: (0, 0)),
            pl.BlockSpec((1, D), lambda i: (0, 0)),
            pl.BlockSpec((D, RH), lambda i: (0, 0)),
            pl.BlockSpec((NC * FB, RH), lambda i: (0, 0)),
            pl.BlockSpec((1, RH), lambda i: (0, 0)),
            pl.BlockSpec((RH, E), lambda i: (0, 0)),
            pl.BlockSpec((1, E), lambda i: (0, 0)),
        ],
        out_specs=[
            pl.BlockSpec((TB, D), lambda i: (i, 0)),
            pl.BlockSpec((TB, 4), lambda i: (i, 0)),
        ],
        out_shape=[
            jax.ShapeDtypeStruct((B, D), jnp.float32),
            jax.ShapeDtypeStruct((B, 4), jnp.float32),
        ],
    )(hidden, stage_feats, ln_gamma.reshape(1, D), ln_beta.reshape(1, D),
      w1h, w1f, rb1.reshape(1, RH), rW2, rb2.reshape(1, E))
    return h_ln, r4


def _gemm_body(te_ref, x_ref, xs_ref, w_ref, w1h_ref, w1f_ref, b1_ref,
               w2_ref, b2_ref, w3_ref, b3_ref, out_ref):
    a = jnp.dot(x_ref[...], w1h_ref[0],
                preferred_element_type=jnp.float32)
    a = a + jnp.dot(xs_ref[...], w1f_ref[0],
                    preferred_element_type=jnp.float32)
    a = _gelu(a + b1_ref[0])
    h2 = _gelu(jnp.dot(a, w2_ref[0],
                       preferred_element_type=jnp.float32) + b2_ref[0])
    o = jnp.dot(h2, w3_ref[0], preferred_element_type=jnp.float32)
    out_ref[...] = (o + b3_ref[0]) * w_ref[...]


def _run_gemm(tile_e, xh, xsf, w_pad, w1f_full, We1, be1, We2, be2, We3,
              be3):
    F2 = NC * FB
    w1h = We1[:, :D, :]
    grid_spec = pltpu.PrefetchScalarGridSpec(
        num_scalar_prefetch=1,
        grid=(MAX_TILES,),
        in_specs=[
            pl.BlockSpec((T, D), lambda i, s: (i, 0)),
            pl.BlockSpec((T, F2), lambda i, s: (i, 0)),
            pl.BlockSpec((T, 1), lambda i, s: (i, 0)),
            pl.BlockSpec((1, D, H), lambda i, s: (s[i], 0, 0)),
            pl.BlockSpec((1, F2, H), lambda i, s: (s[i], 0, 0)),
            pl.BlockSpec((1, 1, H), lambda i, s: (s[i], 0, 0)),
            pl.BlockSpec((1, H, H), lambda i, s: (s[i], 0, 0)),
            pl.BlockSpec((1, 1, H), lambda i, s: (s[i], 0, 0)),
            pl.BlockSpec((1, H, D), lambda i, s: (s[i], 0, 0)),
            pl.BlockSpec((1, 1, D), lambda i, s: (s[i], 0, 0)),
        ],
        out_specs=pl.BlockSpec((T, D), lambda i, s: (i, 0)),
    )
    out_pad = pl.pallas_call(
        _gemm_body,
        grid_spec=grid_spec,
        out_shape=jax.ShapeDtypeStruct((P, D), jnp.float32),
    )(tile_e, xh, xsf, w_pad, w1h, w1f_full, be1.reshape(E, 1, H), We2,
      be2.reshape(E, 1, H), We3, be3.reshape(E, 1, D))
    return out_pad


NWORK = 32
_RPW = P // NWORK
_GCH = 8
_TPW = B // NWORK
_CCH = 8

_SC_MESH = plsc.VectorSubcoreMesh(core_axis_name="c", subcore_axis_name="s")


def _worker_id():
    return lax.axis_index("s") * 2 + lax.axis_index("c")


_FW = NC * FB
_SCH = 16
_SNCH = (B // NWORK) // _SCH


def _sc_dispatch_body(hln, sf, p1r, p2r, xh, xsf, p1v, p2v,
                      hb0, hb1, fbb0, fbb1,
                      g0, g1, g2, g3,
                      s0, s1, s2, s3, s4, s5, s6, s7):
    wid = _worker_id()
    tokbase = wid * _TPW
    pltpu.sync_copy(p1r.at[wid], p1v)
    pltpu.sync_copy(p2r.at[wid], p2v)
    hbufs = [hb0, hb1]
    fbufs = [fbb0, fbb1]
    gsems = [g0, g1, g2, g3]
    ssems = [s0, s1, s2, s3, s4, s5, s6, s7]
    rd = [None, None]
    st = [None, None]

    def start_read(c):
        d = c & 1
        src = pl.ds(tokbase + c * _SCH, _SCH)
        rd[d] = (pltpu.async_copy(hln.at[src], hbufs[d], gsems[d]),
                 pltpu.async_copy(sf.at[src], fbufs[d], gsems[2 + d]))

    start_read(0)
    for c in range(_SNCH):
        d = c & 1
        if c + 1 < _SNCH:
            if st[1 - d] is not None:
                for cp in st[1 - d]:
                    cp.wait()
            start_read(c + 1)
        for cp in rd[d]:
            cp.wait()
        i1 = p1v.at[c]
        i2 = p2v.at[c]
        st[d] = (pltpu.async_copy(hbufs[d], xh.at[i1], ssems[4 * d]),
                 pltpu.async_copy(hbufs[d], xh.at[i2], ssems[4 * d + 1]),
                 pltpu.async_copy(fbufs[d], xsf.at[i1], ssems[4 * d + 2]),
                 pltpu.async_copy(fbufs[d], xsf.at[i2], ssems[4 * d + 3]))
    for cps in st:
        if cps is not None:
            for cp in cps:
                cp.wait()


def _run_sc_dispatch(hln, sf, p1, p2):
    out_type = [
        jax.ShapeDtypeStruct((P, D), jnp.float32),
        jax.ShapeDtypeStruct((P, _FW), jnp.float32),
    ]
    scratch = (
        [pltpu.VMEM((_SNCH, _SCH), jnp.int32)] * 2
        + [pltpu.VMEM((_SCH, D), jnp.float32)] * 2
        + [pltpu.VMEM((_SCH, _FW), jnp.float32)] * 2
        + [pltpu.SemaphoreType.DMA] * 12
    )
    call = pl.kernel(_sc_dispatch_body, out_type=out_type, mesh=_SC_MESH,
                     scratch_types=scratch)
    return call(hln, sf, p1.reshape(NWORK, _SNCH, _SCH),
                p2.reshape(NWORK, _SNCH, _SCH))


def _sc_combine_body(hid, outp, p1, p2, y, p1_v, p2_v,
                     a0, a1, b0, b1, h0, h1,
                     sa0, sa1, sb0, sb1, sh0, sh1, sy0, sy1):
    wid = _worker_id()
    base = wid * _TPW
    pltpu.sync_copy(p1.at[pl.ds(base, _TPW)], p1_v)
    pltpu.sync_copy(p2.at[pl.ds(base, _TPW)], p2_v)
    A = [a0, a1]
    Bb = [b0, b1]
    Hh = [h0, h1]
    SA = [sa0, sa1]
    SB = [sb0, sb1]
    SH = [sh0, sh1]
    SY = [sy0, sy1]
    nch = _TPW // _CCH
    gh = [None, None]
    st = [None, None]

    def start_gathers(c):
        d = c & 1
        s = pl.ds(c * _CCH, _CCH)
        gh[d] = (
            pltpu.async_copy(outp.at[p1_v.at[s]], A[d], SA[d]),
            pltpu.async_copy(outp.at[p2_v.at[s]], Bb[d], SB[d]),
            pltpu.async_copy(hid.at[pl.ds(base + c * _CCH, _CCH)],
                             Hh[d], SH[d]),
        )

    start_gathers(0)
    for c in range(nch):
        d = c & 1
        if c + 1 < nch:
            if st[1 - d] is not None:
                st[1 - d].wait()
            start_gathers(c + 1)
        for cp in gh[d]:
            cp.wait()

        def vbody(i, carry):
            for r in range(_CCH):
                s = pl.ds(i * 16, 16)
                Hh[d][r, s] = Hh[d][r, s] + A[d][r, s] + Bb[d][r, s]
            return carry

        lax.fori_loop(0, D // 16, vbody, 0)
        st[d] = pltpu.async_copy(Hh[d], y.at[pl.ds(base + c * _CCH, _CCH)],
                                 SY[d])
    st[0].wait()
    st[1].wait()


def _run_sc_combine(hidden, out_pad, p1, p2):
    out_type = jax.ShapeDtypeStruct((B, D), jnp.float32)
    scratch = [
        pltpu.VMEM((_TPW,), jnp.int32),
        pltpu.VMEM((_TPW,), jnp.int32),
        pltpu.VMEM((_CCH, D), jnp.float32),
        pltpu.VMEM((_CCH, D), jnp.float32),
        pltpu.VMEM((_CCH, D), jnp.float32),
        pltpu.VMEM((_CCH, D), jnp.float32),
        pltpu.VMEM((_CCH, D), jnp.float32),
        pltpu.VMEM((_CCH, D), jnp.float32),
        pltpu.SemaphoreType.DMA,
        pltpu.SemaphoreType.DMA,
        pltpu.SemaphoreType.DMA,
        pltpu.SemaphoreType.DMA,
        pltpu.SemaphoreType.DMA,
        pltpu.SemaphoreType.DMA,
        pltpu.SemaphoreType.DMA,
        pltpu.SemaphoreType.DMA,
    ]
    call = pl.kernel(_sc_combine_body, out_type=out_type, mesh=_SC_MESH,
                     scratch_types=scratch)
    return call(hidden, out_pad, p1, p2)


def kernel(hidden, feature_bank, expert_bank_idx, ln_gamma, ln_beta,
           rW1, rb1, rW2, rb2, We1, be1, We2, be2, We3, be3, alpha):
    stage_feats = feature_bank.reshape(B, NC * FB)
    hcat, r4 = _run_router(hidden, stage_feats, ln_gamma, ln_beta,
                           rW1, rb1, rW2, rb2)

    i1 = r4[:, 0].astype(jnp.int32)
    i2 = r4[:, 1].astype(jnp.int32)
    e_pair = jnp.stack([i1, i2], axis=1).reshape(-1)
    w_pair = (r4[:, 2:4] * alpha).reshape(-1)
    oh = (e_pair[:, None] == jnp.arange(E)[None, :]).astype(jnp.int32)
    ranks = jnp.cumsum(oh, axis=0)
    rank_in = jnp.take_along_axis(ranks, e_pair[:, None], axis=1)[:, 0] - 1
    counts = ranks[-1]
    tiles_pe = (counts + T - 1) // T
    tile_end = jnp.cumsum(tiles_pe)
    pad_start = (tile_end - tiles_pe) * T
    pos = pad_start[e_pair] + rank_in
    w_pad = jnp.zeros((P,), jnp.float32).at[pos].set(w_pair)
    n_tiles = tile_end[E - 1]
    last_e = jnp.max(jnp.where(counts > 0, jnp.arange(E), 0))
    tile_e = jnp.minimum(
        jnp.sum(jnp.arange(MAX_TILES)[:, None] >= tile_end[None, :], axis=1),
        last_e).astype(jnp.int32)
    p1 = pos[0::2]
    p2 = pos[1::2]

    xh, xsf = _run_sc_dispatch(hcat, stage_feats,
                               p1.astype(jnp.int32), p2.astype(jnp.int32))

    w1f = We1[:, D:, :].reshape(E, FPE, FB, H)
    col = expert_bank_idx.astype(jnp.int32)
    w1f_full = jnp.zeros((E, NC, FB, H), jnp.float32).at[
        jnp.arange(E)[:, None], col].set(w1f).reshape(E, NC * FB, H)

    tile_meta = jnp.concatenate([tile_e, n_tiles[None].astype(jnp.int32)])
    out_pad = _run_gemm(tile_meta, xh, xsf, w_pad.reshape(P, 1), w1f_full,
                        We1, be1, We2, be2, We3, be3)

    y = _run_sc_combine(hidden, out_pad, p1.astype(jnp.int32),
                        p2.astype(jnp.int32))
    return y


# speedup vs baseline: 1.0018x; 1.0018x over previous
import jax
import jax.numpy as jnp
from jax import lax
from jax.experimental import pallas as pl
from jax.experimental.pallas import tpu as pltpu
from jax.experimental.pallas import tpu_sc as plsc

B = 2048
D = 2048
E = 8
NC = 16
FB = 16
FPE = 2
H = 1024
RH = 1024
K = 2

T = 256
MAX_TILES = 23
P = MAX_TILES * T


_SQRT_HALF = 0.7071067811865476


def _gelu(x):
    return 0.5 * x * (1.0 + jax.lax.erf(x * _SQRT_HALF))


def _router_body(hid_ref, sf_ref, g_ref, b_ref, w1h_ref, w1f_ref, b1_ref,
                 w2_ref, b2_ref, h_out_ref, r4_ref):
    x = hid_ref[...]
    mu = jnp.mean(x, axis=1, keepdims=True)
    xc = x - mu
    var = jnp.mean(xc * xc, axis=1, keepdims=True)
    h = xc * jax.lax.rsqrt(var + 1e-5) * g_ref[...] + b_ref[...]
    h_out_ref[...] = h
    t1 = jnp.dot(h, w1h_ref[...], preferred_element_type=jnp.float32)
    t1 = t1 + jnp.dot(sf_ref[...], w1f_ref[...],
                      preferred_element_type=jnp.float32)
    t1 = _gelu(t1 + b1_ref[...])
    logits = jnp.dot(t1, w2_ref[...],
                     preferred_element_type=jnp.float32) + b2_ref[...]
    i1 = jnp.argmax(logits, axis=1)
    v1 = jnp.max(logits, axis=1, keepdims=True)
    masked = jnp.where(jnp.arange(E)[None, :] == i1[:, None],
                       -jnp.inf, logits)
    i2 = jnp.argmax(masked, axis=1)
    v2 = jnp.max(masked, axis=1, keepdims=True)
    e2 = jnp.exp(v2 - v1)
    w1 = 1.0 / (1.0 + e2)
    w2 = e2 * w1
    r4_ref[...] = jnp.concatenate(
        [i1[:, None].astype(jnp.float32), i2[:, None].astype(jnp.float32),
         w1, w2], axis=1)


def _run_router(hidden, stage_feats, ln_gamma, ln_beta, rW1, rb1, rW2, rb2):
    TB = 256
    grid = (B // TB,)
    w1h = rW1[:D]
    w1f = rW1[D:]
    h_ln, r4 = pl.pallas_call(
        _router_body,
        grid=grid,
        in_specs=[
            pl.BlockSpec((TB, D), lambda i: (i, 0)),
            pl.BlockSpec((TB, NC * FB), lambda i: (i, 0)),
            pl.BlockSpec((1, D), lambda i: (0, 0)),
            pl.BlockSpec((1, D), lambda i: (0, 0)),
            pl.BlockSpec((D, RH), lambda i: (0, 0)),
            pl.BlockSpec((NC * FB, RH), lambda i: (0, 0)),
            pl.BlockSpec((1, RH), lambda i: (0, 0)),
            pl.BlockSpec((RH, E), lambda i: (0, 0)),
            pl.BlockSpec((1, E), lambda i: (0, 0)),
        ],
        out_specs=[
            pl.BlockSpec((TB, D), lambda i: (i, 0)),
            pl.BlockSpec((TB, 4), lambda i: (i, 0)),
        ],
        out_shape=[
            jax.ShapeDtypeStruct((B, D), jnp.float32),
            jax.ShapeDtypeStruct((B, 4), jnp.float32),
        ],
    )(hidden, stage_feats, ln_gamma.reshape(1, D), ln_beta.reshape(1, D),
      w1h, w1f, rb1.reshape(1, RH), rW2, rb2.reshape(1, E))
    return h_ln, r4


def _gemm_body(te_ref, x_ref, xs_ref, w_ref, w1h_ref, w1f_ref, b1_ref,
               w2_ref, b2_ref, w3_ref, b3_ref, out_ref):
    a = jnp.dot(x_ref[...], w1h_ref[0],
                preferred_element_type=jnp.float32)
    a = a + jnp.dot(xs_ref[...], w1f_ref[0],
                    preferred_element_type=jnp.float32)
    a = _gelu(a + b1_ref[0])
    h2 = _gelu(jnp.dot(a, w2_ref[0],
                       preferred_element_type=jnp.float32) + b2_ref[0])
    o = jnp.dot(h2, w3_ref[0], preferred_element_type=jnp.float32)
    out_ref[...] = (o + b3_ref[0]) * w_ref[...]


def _run_gemm(tile_e, xh, xsf, w_pad, w1f_full, We1, be1, We2, be2, We3,
              be3):
    F2 = NC * FB
    w1h = We1[:, :D, :]
    grid_spec = pltpu.PrefetchScalarGridSpec(
        num_scalar_prefetch=1,
        grid=(MAX_TILES,),
        in_specs=[
            pl.BlockSpec((T, D), lambda i, s: (i, 0)),
            pl.BlockSpec((T, F2), lambda i, s: (i, 0)),
            pl.BlockSpec((T, 1), lambda i, s: (i, 0)),
            pl.BlockSpec((1, D, H), lambda i, s: (s[i], 0, 0)),
            pl.BlockSpec((1, F2, H), lambda i, s: (s[i], 0, 0)),
            pl.BlockSpec((1, 1, H), lambda i, s: (s[i], 0, 0)),
            pl.BlockSpec((1, H, H), lambda i, s: (s[i], 0, 0)),
            pl.BlockSpec((1, 1, H), lambda i, s: (s[i], 0, 0)),
            pl.BlockSpec((1, H, D), lambda i, s: (s[i], 0, 0)),
            pl.BlockSpec((1, 1, D), lambda i, s: (s[i], 0, 0)),
        ],
        out_specs=pl.BlockSpec((T, D), lambda i, s: (i, 0)),
    )
    out_pad = pl.pallas_call(
        _gemm_body,
        grid_spec=grid_spec,
        out_shape=jax.ShapeDtypeStruct((P, D), jnp.float32),
    )(tile_e, xh, xsf, w_pad, w1h, w1f_full, be1.reshape(E, 1, H), We2,
      be2.reshape(E, 1, H), We3, be3.reshape(E, 1, D))
    return out_pad


NWORK = 32
_TPW = B // NWORK
_CCH = 8

_SC_MESH = plsc.VectorSubcoreMesh(core_axis_name="c", subcore_axis_name="s")


def _worker_id():
    return lax.axis_index("s") * 2 + lax.axis_index("c")


_FW = NC * FB
_SCH = 16
_SNCH = (B // NWORK) // _SCH


def _sc_dispatch_body(hln, sf, p1r, p2r, xh, xsf, p1v, p2v,
                      hb0, hb1, fbb0, fbb1,
                      g0, g1, g2, g3,
                      s0, s1, s2, s3, s4, s5, s6, s7):
    wid = _worker_id()
    tokbase = wid * _TPW
    pltpu.sync_copy(p1r.at[wid], p1v)
    pltpu.sync_copy(p2r.at[wid], p2v)
    hbufs = [hb0, hb1]
    fbufs = [fbb0, fbb1]
    gsems = [g0, g1, g2, g3]
    ssems = [s0, s1, s2, s3, s4, s5, s6, s7]
    rd = [None, None]
    st = [None, None]

    def start_read(c):
        d = c & 1
        src = pl.ds(tokbase + c * _SCH, _SCH)
        rd[d] = (pltpu.async_copy(hln.at[src], hbufs[d], gsems[d]),
                 pltpu.async_copy(sf.at[src], fbufs[d], gsems[2 + d]))

    start_read(0)
    for c in range(_SNCH):
        d = c & 1
        if c + 1 < _SNCH:
            if st[1 - d] is not None:
                for cp in st[1 - d]:
                    cp.wait()
            start_read(c + 1)
        for cp in rd[d]:
            cp.wait()
        i1 = p1v.at[c]
        i2 = p2v.at[c]
        st[d] = (pltpu.async_copy(hbufs[d], xh.at[i1], ssems[4 * d]),
                 pltpu.async_copy(hbufs[d], xh.at[i2], ssems[4 * d + 1]),
                 pltpu.async_copy(fbufs[d], xsf.at[i1], ssems[4 * d + 2]),
                 pltpu.async_copy(fbufs[d], xsf.at[i2], ssems[4 * d + 3]))
    for cps in st:
        if cps is not None:
            for cp in cps:
                cp.wait()


def _run_sc_dispatch(hln, sf, p1, p2):
    out_type = [
        jax.ShapeDtypeStruct((P, D), jnp.float32),
        jax.ShapeDtypeStruct((P, _FW), jnp.float32),
    ]
    scratch = (
        [pltpu.VMEM((_SNCH, _SCH), jnp.int32)] * 2
        + [pltpu.VMEM((_SCH, D), jnp.float32)] * 2
        + [pltpu.VMEM((_SCH, _FW), jnp.float32)] * 2
        + [pltpu.SemaphoreType.DMA] * 12
    )
    call = pl.kernel(_sc_dispatch_body, out_type=out_type, mesh=_SC_MESH,
                     scratch_types=scratch)
    return call(hln, sf, p1.reshape(NWORK, _SNCH, _SCH),
                p2.reshape(NWORK, _SNCH, _SCH))


def _sc_combine_body(hid, outp, p1, p2, y, p1_v, p2_v,
                     a0, a1, b0, b1, h0, h1,
                     sa0, sa1, sb0, sb1, sh0, sh1, sy0, sy1):
    wid = _worker_id()
    base = wid * _TPW
    pltpu.sync_copy(p1.at[pl.ds(base, _TPW)], p1_v)
    pltpu.sync_copy(p2.at[pl.ds(base, _TPW)], p2_v)
    A = [a0, a1]
    Bb = [b0, b1]
    Hh = [h0, h1]
    SA = [sa0, sa1]
    SB = [sb0, sb1]
    SH = [sh0, sh1]
    SY = [sy0, sy1]
    nch = _TPW // _CCH
    gh = [None, None]
    st = [None, None]

    def start_gathers(c):
        d = c & 1
        s = pl.ds(c * _CCH, _CCH)
        gh[d] = (
            pltpu.async_copy(outp.at[p1_v.at[s]], A[d], SA[d]),
            pltpu.async_copy(outp.at[p2_v.at[s]], Bb[d], SB[d]),
            pltpu.async_copy(hid.at[pl.ds(base + c * _CCH, _CCH)],
                             Hh[d], SH[d]),
        )

    start_gathers(0)
    for c in range(nch):
        d = c & 1
        if c + 1 < nch:
            if st[1 - d] is not None:
                st[1 - d].wait()
            start_gathers(c + 1)
        for cp in gh[d]:
            cp.wait()

        def vbody(i, carry):
            for r in range(_CCH):
                s = pl.ds(i * 16, 16)
                Hh[d][r, s] = Hh[d][r, s] + A[d][r, s] + Bb[d][r, s]
            return carry

        lax.fori_loop(0, D // 16, vbody, 0)
        st[d] = pltpu.async_copy(Hh[d], y.at[pl.ds(base + c * _CCH, _CCH)],
                                 SY[d])
    st[0].wait()
    st[1].wait()


def _run_sc_combine(hidden, out_pad, p1, p2):
    out_type = jax.ShapeDtypeStruct((B, D), jnp.float32)
    scratch = [
        pltpu.VMEM((_TPW,), jnp.int32),
        pltpu.VMEM((_TPW,), jnp.int32),
        pltpu.VMEM((_CCH, D), jnp.float32),
        pltpu.VMEM((_CCH, D), jnp.float32),
        pltpu.VMEM((_CCH, D), jnp.float32),
        pltpu.VMEM((_CCH, D), jnp.float32),
        pltpu.VMEM((_CCH, D), jnp.float32),
        pltpu.VMEM((_CCH, D), jnp.float32),
        pltpu.SemaphoreType.DMA,
        pltpu.SemaphoreType.DMA,
        pltpu.SemaphoreType.DMA,
        pltpu.SemaphoreType.DMA,
        pltpu.SemaphoreType.DMA,
        pltpu.SemaphoreType.DMA,
        pltpu.SemaphoreType.DMA,
        pltpu.SemaphoreType.DMA,
    ]
    call = pl.kernel(_sc_combine_body, out_type=out_type, mesh=_SC_MESH,
                     scratch_types=scratch)
    return call(hidden, out_pad, p1, p2)


def kernel(hidden, feature_bank, expert_bank_idx, ln_gamma, ln_beta,
           rW1, rb1, rW2, rb2, We1, be1, We2, be2, We3, be3, alpha):
    stage_feats = feature_bank.reshape(B, NC * FB)
    h_ln, r4 = _run_router(hidden, stage_feats, ln_gamma, ln_beta,
                           rW1, rb1, rW2, rb2)

    i1 = r4[:, 0].astype(jnp.int32)
    i2 = r4[:, 1].astype(jnp.int32)
    e_pair = jnp.stack([i1, i2], axis=1).reshape(-1)
    w_pair = (r4[:, 2:4] * alpha).reshape(-1)
    oh = (e_pair[:, None] == jnp.arange(E)[None, :]).astype(jnp.int32)
    ranks = jnp.cumsum(oh, axis=0)
    rank_in = jnp.take_along_axis(ranks, e_pair[:, None], axis=1)[:, 0] - 1
    counts = ranks[-1]
    tiles_pe = (counts + T - 1) // T
    tile_end = jnp.cumsum(tiles_pe)
    pad_start = (tile_end - tiles_pe) * T
    pos = pad_start[e_pair] + rank_in
    w_pad = jnp.zeros((P,), jnp.float32).at[pos].set(w_pair)
    n_tiles = tile_end[E - 1]
    last_e = jnp.max(jnp.where(counts > 0, jnp.arange(E), 0))
    tile_e = jnp.minimum(
        jnp.sum(jnp.arange(MAX_TILES)[:, None] >= tile_end[None, :], axis=1),
        last_e).astype(jnp.int32)
    p1 = pos[0::2]
    p2 = pos[1::2]

    xh, xsf = _run_sc_dispatch(h_ln, stage_feats,
                               p1.astype(jnp.int32), p2.astype(jnp.int32))

    w1f = We1[:, D:, :].reshape(E, FPE, FB, H)
    col = expert_bank_idx.astype(jnp.int32)
    w1f_full = jnp.zeros((E, NC, FB, H), jnp.float32).at[
        jnp.arange(E)[:, None], col].set(w1f).reshape(E, NC * FB, H)

    tile_meta = jnp.concatenate([tile_e, n_tiles[None].astype(jnp.int32)])
    out_pad = _run_gemm(tile_meta, xh, xsf, w_pad.reshape(P, 1), w1f_full,
                        We1, be1, We2, be2, We3, be3)

    y = _run_sc_combine(hidden, out_pad, p1.astype(jnp.int32),
                        p2.astype(jnp.int32))
    return y
